# baseline (speedup 1.0000x reference)
import functools

import jax
import jax.numpy as jnp
from jax import lax
from jax.experimental import pallas as pl
from jax.experimental.pallas import tpu as pltpu
from jax.experimental.pallas import tpu_sc as plsc

N = 4096
E = 16384
NNZ = 131072
D = 128
NPOS = N * N
NC = 2
NS = 16
NW = NC * NS
L = 16

f32 = jnp.float32
i32 = jnp.int32

_MESH = plsc.VectorSubcoreMesh(core_axis_name="c", subcore_axis_name="s")


def _zero_vmem(ref, n):
    def body(i, _):
        ref[pl.ds(i * L, L)] = jnp.zeros((L,), ref.dtype)
        return 0
    lax.fori_loop(0, n // L, body, 0)


def _edge_mlp_body(v_ref, w1_ref, b1_ref, w2_ref, b2_ref, out_ref):
    x = v_ref[...]
    h = jnp.dot(x, w1_ref[...], preferred_element_type=f32) + b1_ref[...]
    h = jnp.maximum(h, 0.0)
    out_ref[...] = jnp.dot(h, w2_ref[...], preferred_element_type=f32) + b2_ref[0, 0]


def _edge_mlp(v_col, w1, b1, w2, b2):
    BM = 4096
    return pl.pallas_call(
        _edge_mlp_body,
        grid=(NNZ // BM,),
        in_specs=[
            pl.BlockSpec((BM, 1), lambda i: (i, 0)),
            pl.BlockSpec((1, D), lambda i: (0, 0)),
            pl.BlockSpec((1, D), lambda i: (0, 0)),
            pl.BlockSpec((D, 1), lambda i: (0, 0)),
            pl.BlockSpec((1, 1), lambda i: (0, 0)),
        ],
        out_specs=pl.BlockSpec((BM, 1), lambda i: (i, 0)),
        out_shape=jax.ShapeDtypeStruct((NNZ, 1), f32),
    )(v_col, w1, b1, w2, b2)


def _node_mlp_body(s_ref, w1_ref, b1_ref, w2_ref, b2_ref, s2_ref):
    x = s_ref[...]
    h = jnp.dot(x, w1_ref[...], preferred_element_type=f32) + b1_ref[...]
    h = jnp.maximum(h, 0.0)
    st = jnp.dot(h, w2_ref[...], preferred_element_type=f32) + b2_ref[0, 0]
    s2_ref[...] = st * st


def _node_mlp_sq(s_col, w1, b1, w2, b2):
    return pl.pallas_call(
        _node_mlp_body,
        out_shape=jax.ShapeDtypeStruct((N, 1), f32),
    )(s_col, w1, b1, w2, b2)


def _edge_dot_body(hs_ref, hd_ref, al_ref, hc_ref, a0_ref):
    ex = jnp.exp(al_ref[...])
    ea0 = ex[0, 0]
    ea1 = ex[0, 1]
    denom = ea0 + ea1
    a1 = ea1 / denom
    dot = jnp.sum(hs_ref[...] * hd_ref[...], axis=1, keepdims=True)
    hc_ref[...] = a1 / (1.0 + jnp.exp(-dot)) + 1e-15
    a0_ref[...] = jnp.full((1, L), ea0 / denom, f32)


def _edge_dot(hs, hd, alpha_pad):
    BE = 2048
    return pl.pallas_call(
        _edge_dot_body,
        grid=(E // BE,),
        in_specs=[
            pl.BlockSpec((BE, D), lambda i: (i, 0)),
            pl.BlockSpec((BE, D), lambda i: (i, 0)),
            pl.BlockSpec((1, D), lambda i: (0, 0)),
        ],
        out_specs=[
            pl.BlockSpec((BE, 1), lambda i: (i, 0)),
            pl.BlockSpec((1, L), lambda i: (0, 0)),
        ],
        out_shape=[
            jax.ShapeDtypeStruct((E, 1), f32),
            jax.ShapeDtypeStruct((1, L), f32),
        ],
    )(hs, hd, alpha_pad)


CH_B = NNZ // NW
EC_B = E // NW


def _sc_scatter_gather_body(row_h, col_h, src_h, dst_h, H_h,
                            T_h, Hs_h, Hd_h,
                            rbuf, cbuf, pbuf, kbuf, sibuf, dibuf, hbuf, hbuf2,
                            sem, sem2):
    c = lax.axis_index("c")
    s = lax.axis_index("s")
    wid = s * NC + c
    base = wid * CH_B
    pltpu.sync_copy(row_h.at[pl.ds(base, CH_B)], rbuf)
    pltpu.sync_copy(col_h.at[pl.ds(base, CH_B)], cbuf)

    def body(i, _):
        rr = rbuf[pl.ds(i * L, L)]
        cc = cbuf[pl.ds(i * L, L)]
        pbuf[pl.ds(i * L, L)] = rr * N + cc
        kbuf[pl.ds(i * L, L)] = lax.iota(i32, L) + (base + i * L)
        return 0

    lax.fori_loop(0, CH_B // L, body, 0)
    tcopy = pltpu.async_copy(kbuf, T_h.at[pbuf], sem2)

    EH = EC_B // 2
    ebase = wid * EC_B
    pltpu.sync_copy(src_h.at[pl.ds(ebase, EC_B)], sibuf)
    pltpu.sync_copy(dst_h.at[pl.ds(ebase, EC_B)], dibuf)
    for idxbuf, out_h in ((sibuf, Hs_h), (dibuf, Hd_h)):
        g1 = pltpu.async_copy(H_h.at[idxbuf.at[pl.ds(0, EH)]], hbuf, sem)
        g2 = pltpu.async_copy(H_h.at[idxbuf.at[pl.ds(EH, EH)]], hbuf2, sem)
        g1.wait()
        pltpu.sync_copy(hbuf, out_h.at[pl.ds(ebase, EH)])
        g2.wait()
        pltpu.sync_copy(hbuf2, out_h.at[pl.ds(ebase + EH, EH)])
    tcopy.wait()


def _sc_scatter_gather(row, col, src, dst, H):
    return pl.kernel(
        _sc_scatter_gather_body,
        out_type=(
            jax.ShapeDtypeStruct((NPOS,), i32),
            jax.ShapeDtypeStruct((E, D), f32),
            jax.ShapeDtypeStruct((E, D), f32),
        ),
        mesh=_MESH,
        name="sc1_scatter_gather",
        scratch_types=[
            pltpu.VMEM((CH_B,), i32),
            pltpu.VMEM((CH_B,), i32),
            pltpu.VMEM((CH_B,), i32),
            pltpu.VMEM((CH_B,), i32),
            pltpu.VMEM((EC_B,), i32),
            pltpu.VMEM((EC_B,), i32),
            pltpu.VMEM((EC_B // 2, D), f32),
            pltpu.VMEM((EC_B // 2, D), f32),
            pltpu.SemaphoreType.DMA,
            pltpu.SemaphoreType.DMA,
        ],
    )(row, col, src, dst, H)


CH_D = NNZ // NS
ZB = 2048
NPS = N // NS


def _sc_dedup_body(row_h, col_h, v_h, e_h, T_h,
                   u_h, s_h,
                   rbuf, cbuf, pbuf, wbuf, vbuf, ubuf, zbuf, acc, sacc, sem):
    c = lax.axis_index("c")
    sid = lax.axis_index("s")
    base = sid * CH_D
    _zero_vmem(zbuf, ZB)

    @pl.when(c == 0)
    def _():
        for j in range(CH_D // ZB):
            pltpu.sync_copy(zbuf, acc.at[pl.ds(base + j * ZB, ZB)])
        pltpu.sync_copy(row_h.at[pl.ds(base, CH_D)], rbuf)
        pltpu.sync_copy(col_h.at[pl.ds(base, CH_D)], cbuf)

        def body(i, _):
            pbuf[pl.ds(i * L, L)] = rbuf[pl.ds(i * L, L)] * N + cbuf[pl.ds(i * L, L)]
            return 0

        lax.fori_loop(0, CH_D // L, body, 0)
        pltpu.async_copy(T_h.at[pbuf], wbuf, sem).wait()

        def clamp(i, _):
            sl = pl.ds(i * L, L)
            wbuf[sl] = lax.bitwise_and(wbuf[sl], NNZ - 1)
            return 0

        lax.fori_loop(0, CH_D // L, clamp, 0)
        pltpu.sync_copy(v_h.at[pl.ds(base, CH_D)], vbuf)

    @pl.when(c == 1)
    def _():
        pltpu.sync_copy(zbuf.at[pl.ds(0, NPS)], sacc.at[pl.ds(sid * NPS, NPS)])
        pltpu.sync_copy(e_h.at[pl.ds(base, CH_D)], vbuf)
        pltpu.sync_copy(col_h.at[pl.ds(base, CH_D)], cbuf)

    plsc.subcore_barrier()

    @pl.when(c == 0)
    def _():
        pltpu.sync_copy(vbuf, acc.at[wbuf], add=True)

    @pl.when(c == 1)
    def _():
        pltpu.sync_copy(vbuf, sacc.at[cbuf], add=True)

    plsc.subcore_barrier()

    @pl.when(c == 0)
    def _():
        pltpu.async_copy(acc.at[wbuf], ubuf, sem).wait()
        pltpu.sync_copy(ubuf, u_h.at[pl.ds(base, CH_D)])

    @pl.when(c == 1)
    def _():
        pltpu.sync_copy(sacc.at[pl.ds(sid * NPS, NPS)], s_h.at[pl.ds(sid * NPS, NPS)])


def _sc_dedup(row, col, v, e_flat, T):
    return pl.kernel(
        _sc_dedup_body,
        out_type=(
            jax.ShapeDtypeStruct((NNZ,), f32),
            jax.ShapeDtypeStruct((N,), f32),
        ),
        mesh=_MESH,
        name="sc2_dedup",
        scratch_types=[
            pltpu.VMEM((CH_D,), i32),
            pltpu.VMEM((CH_D,), i32),
            pltpu.VMEM((CH_D,), i32),
            pltpu.VMEM((CH_D,), i32),
            pltpu.VMEM((CH_D,), f32),
            pltpu.VMEM((CH_D,), f32),
            pltpu.VMEM((ZB,), f32),
            pltpu.VMEM_SHARED((NNZ,), f32),
            pltpu.VMEM_SHARED((N,), f32),
            pltpu.SemaphoreType.DMA,
        ],
    )(row, col, v, e_flat, T)


EPT = E // NS
NG = NPS // L


def _sc_final_body(row_h, col_h, v_h, u_h, s2_h, src_h, hc_h, a0_h,
                   pw1_h, pb1_h, pw2_h, pb2_h,
                   out_h,
                   rbuf, cbuf, vbuf, ubuf, dbuf, s2gbuf, zerobuf,
                   tbuf, znbuf, w1buf, b1buf, w2buf, b2buf, a0buf,
                   sbuf, hcbuf, obuf, zgbuf, s2sp, tacc, zsp, sem):
    c = lax.axis_index("c")
    sid = lax.axis_index("s")
    base = sid * CH_D
    _zero_vmem(zerobuf, ZB)

    @pl.when(c == 0)
    def _():
        pltpu.sync_copy(zerobuf.at[pl.ds(0, NPS)], tacc.at[pl.ds(sid * NPS, NPS)])
        pltpu.sync_copy(s2_h.at[pl.ds(sid * NPS, NPS)], s2sp.at[pl.ds(sid * NPS, NPS)])
        pltpu.sync_copy(row_h.at[pl.ds(base, CH_D)], rbuf)
        pltpu.sync_copy(col_h.at[pl.ds(base, CH_D)], cbuf)
        pltpu.sync_copy(v_h.at[pl.ds(base, CH_D)], vbuf)
        pltpu.sync_copy(u_h.at[pl.ds(base, CH_D)], ubuf)

    plsc.subcore_barrier()

    @pl.when(c == 0)
    def _():
        pltpu.async_copy(s2sp.at[cbuf], s2gbuf, sem).wait()

        def body(i, _):
            sl = pl.ds(i * L, L)
            dbuf[sl] = vbuf[sl] * ubuf[sl] * s2gbuf[sl]
            return 0

        lax.fori_loop(0, CH_D // L, body, 0)
        pltpu.sync_copy(dbuf, tacc.at[rbuf], add=True)

    plsc.subcore_barrier()

    @pl.when(c == 0)
    def _():
        pltpu.sync_copy(tacc.at[pl.ds(sid * NPS, NPS)], tbuf)
        pltpu.sync_copy(pw1_h, w1buf)
        pltpu.sync_copy(pb1_h, b1buf)
        pltpu.sync_copy(pw2_h, w2buf)
        pltpu.sync_copy(pb2_h, b2buf)
        tv = [tbuf[pl.ds(g * L, L)] for g in range(NG)]
        accs = [jnp.zeros((L,), f32) for _ in range(NG)]

        def hbody(h, accs):
            w1h = w1buf[pl.ds(h, L)][0]
            b1h = b1buf[pl.ds(h, L)][0]
            w2h = w2buf[pl.ds(h, L)][0]
            return tuple(
                a + jnp.maximum(t * w1h + b1h, 0.0) * w2h
                for a, t in zip(accs, tv)
            )

        accs = lax.fori_loop(0, D, hbody, tuple(accs))
        b2v = b2buf[...]
        for g in range(NG):
            z = 1.0 / (1.0 + jnp.exp(-(accs[g] + b2v)))
            znbuf[pl.ds(g * L, L)] = z
        pltpu.sync_copy(znbuf, zsp.at[pl.ds(sid * NPS, NPS)])

    plsc.subcore_barrier()

    @pl.when(c == 0)
    def _():
        ebase = sid * EPT
        pltpu.sync_copy(src_h.at[pl.ds(ebase, EPT)], sbuf)
        pltpu.sync_copy(hc_h.at[pl.ds(ebase, EPT)], hcbuf)
        pltpu.sync_copy(a0_h, a0buf)
        pltpu.async_copy(zsp.at[sbuf], zgbuf, sem).wait()
        a0v = a0buf[...]

        def ebody(i, _):
            sl = pl.ds(i * L, L)
            obuf[sl] = a0v * zgbuf[sl] + hcbuf[sl]
            return 0

        lax.fori_loop(0, EPT // L, ebody, 0)
        pltpu.sync_copy(obuf, out_h.at[pl.ds(ebase, EPT)])


def _sc_final(row, col, v, u, s2, src, hc, a0vec, pw1, pb1, pw2, pb2):
    return pl.kernel(
        _sc_final_body,
        out_type=jax.ShapeDtypeStruct((E,), f32),
        mesh=_MESH,
        name="sc3_final",
        scratch_types=[
            pltpu.VMEM((CH_D,), i32),
            pltpu.VMEM((CH_D,), i32),
            pltpu.VMEM((CH_D,), f32),
            pltpu.VMEM((CH_D,), f32),
            pltpu.VMEM((CH_D,), f32),
            pltpu.VMEM((CH_D,), f32),
            pltpu.VMEM((ZB,), f32),
            pltpu.VMEM((NPS,), f32),
            pltpu.VMEM((NPS,), f32),
            pltpu.VMEM((D + L,), f32),
            pltpu.VMEM((D + L,), f32),
            pltpu.VMEM((D + L,), f32),
            pltpu.VMEM((L,), f32),
            pltpu.VMEM((L,), f32),
            pltpu.VMEM((EPT,), i32),
            pltpu.VMEM((EPT,), f32),
            pltpu.VMEM((EPT,), f32),
            pltpu.VMEM((EPT,), f32),
            pltpu.VMEM_SHARED((N,), f32),
            pltpu.VMEM_SHARED((N,), f32),
            pltpu.VMEM_SHARED((N,), f32),
            pltpu.SemaphoreType.DMA,
        ],
    )(row, col, v, u, s2, src, hc, a0vec, pw1, pb1, pw2, pb2)


def kernel(edges, H, overlap_row, overlap_col, overlap_values,
           f_edge_w1, f_edge_b1, f_edge_w2, f_edge_b2,
           f_node_w1, f_node_b1, f_node_w2, f_node_b2,
           psi3_w1, psi3_b1, psi3_w2, psi3_b2, alpha):
    src = edges[:, 0].astype(i32)
    dst = edges[:, 1].astype(i32)
    row = overlap_row.astype(i32)
    col = overlap_col.astype(i32)
    v = overlap_values.astype(f32)

    e = _edge_mlp(v[:, None], f_edge_w1, f_edge_b1[None, :], f_edge_w2,
                  f_edge_b2[None, :])[:, 0]

    T, hs, hd = _sc_scatter_gather(row, col, src, dst, H)

    u, s = _sc_dedup(row, col, v, e, T)

    alpha_pad = jnp.zeros((1, D), f32).at[0, :2].set(alpha.astype(f32))
    hc2, a0vec2 = _edge_dot(hs, hd, alpha_pad)
    hc = hc2[:, 0]
    a0vec = a0vec2[0]

    s2 = _node_mlp_sq(s[:, None], f_node_w1, f_node_b1[None, :], f_node_w2,
                      f_node_b2[None, :])[:, 0]

    pad = lambda x: jnp.pad(x.reshape(D), (0, L))
    out = _sc_final(row, col, v, u, s2, src, hc, a0vec,
                    pad(psi3_w1), pad(psi3_b1), pad(psi3_w2),
                    jnp.broadcast_to(psi3_b2, (L,)))

    return out[:, None]


# speedup vs baseline: 1.0186x; 1.0186x over previous
import functools

import jax
import jax.numpy as jnp
from jax import lax
from jax.experimental import pallas as pl
from jax.experimental.pallas import tpu as pltpu
from jax.experimental.pallas import tpu_sc as plsc

N = 4096
E = 16384
NNZ = 131072
D = 128
NPOS = N * N
NC = 2
NS = 16
NW = NC * NS
L = 16

f32 = jnp.float32
i32 = jnp.int32

_MESH = plsc.VectorSubcoreMesh(core_axis_name="c", subcore_axis_name="s")


def _zero_vmem(ref, n):
    def body(i, _):
        ref[pl.ds(i * L, L)] = jnp.zeros((L,), ref.dtype)
        return 0
    lax.fori_loop(0, n // L, body, 0)


def _edge_mlp_body(v_ref, w1_ref, b1_ref, w2_ref, b2_ref, out_ref):
    x = v_ref[...]
    h = jnp.dot(x, w1_ref[...], preferred_element_type=f32) + b1_ref[...]
    h = jnp.maximum(h, 0.0)
    out_ref[...] = jnp.dot(h, w2_ref[...], preferred_element_type=f32) + b2_ref[0, 0]


def _edge_mlp(v_col, w1, b1, w2, b2):
    BM = 4096
    return pl.pallas_call(
        _edge_mlp_body,
        grid=(NNZ // BM,),
        in_specs=[
            pl.BlockSpec((BM, 1), lambda i: (i, 0)),
            pl.BlockSpec((1, D), lambda i: (0, 0)),
            pl.BlockSpec((1, D), lambda i: (0, 0)),
            pl.BlockSpec((D, 1), lambda i: (0, 0)),
            pl.BlockSpec((1, 1), lambda i: (0, 0)),
        ],
        out_specs=pl.BlockSpec((BM, 1), lambda i: (i, 0)),
        out_shape=jax.ShapeDtypeStruct((NNZ, 1), f32),
    )(v_col, w1, b1, w2, b2)


def _node_mlp_body(s_ref, w1_ref, b1_ref, w2_ref, b2_ref, s2_ref):
    x = s_ref[...]
    h = jnp.dot(x, w1_ref[...], preferred_element_type=f32) + b1_ref[...]
    h = jnp.maximum(h, 0.0)
    st = jnp.dot(h, w2_ref[...], preferred_element_type=f32) + b2_ref[0, 0]
    s2_ref[...] = st * st


def _node_mlp_sq(s_col, w1, b1, w2, b2):
    return pl.pallas_call(
        _node_mlp_body,
        out_shape=jax.ShapeDtypeStruct((N, 1), f32),
    )(s_col, w1, b1, w2, b2)


def _edge_dot_body(hs_ref, hd_ref, al_ref, hc_ref, a0_ref):
    ex = jnp.exp(al_ref[...])
    ea0 = ex[0, 0]
    ea1 = ex[0, 1]
    denom = ea0 + ea1
    a1 = ea1 / denom
    dot = jnp.sum(hs_ref[...] * hd_ref[...], axis=1, keepdims=True)
    hc_ref[...] = a1 / (1.0 + jnp.exp(-dot)) + 1e-15
    a0_ref[...] = jnp.full((1, L), ea0 / denom, f32)


def _edge_dot(hs, hd, alpha_pad):
    BE = 2048
    return pl.pallas_call(
        _edge_dot_body,
        grid=(E // BE,),
        in_specs=[
            pl.BlockSpec((BE, D), lambda i: (i, 0)),
            pl.BlockSpec((BE, D), lambda i: (i, 0)),
            pl.BlockSpec((1, D), lambda i: (0, 0)),
        ],
        out_specs=[
            pl.BlockSpec((BE, 1), lambda i: (i, 0)),
            pl.BlockSpec((1, L), lambda i: (0, 0)),
        ],
        out_shape=[
            jax.ShapeDtypeStruct((E, 1), f32),
            jax.ShapeDtypeStruct((1, L), f32),
        ],
    )(hs, hd, alpha_pad)


CH_B = NNZ // NW
EC_B = E // NW


def _sc_scatter_gather_body(row_h, col_h, src_h, dst_h, H_h,
                            T_h, Hs_h, Hd_h,
                            rbuf, cbuf, pbuf, kbuf, sibuf, dibuf, hbuf, hbuf2,
                            sem, sem2):
    c = lax.axis_index("c")
    s = lax.axis_index("s")
    wid = s * NC + c
    base = wid * CH_B
    pltpu.sync_copy(row_h.at[pl.ds(base, CH_B)], rbuf)
    pltpu.sync_copy(col_h.at[pl.ds(base, CH_B)], cbuf)

    def body(i, _):
        rr = rbuf[pl.ds(i * L, L)]
        cc = cbuf[pl.ds(i * L, L)]
        pbuf[pl.ds(i * L, L)] = rr * N + cc
        kbuf[pl.ds(i * L, L)] = lax.iota(i32, L) + (base + i * L)
        return 0

    lax.fori_loop(0, CH_B // L, body, 0)
    pltpu.sync_copy(kbuf, T_h.at[pbuf])

    EH = EC_B // 2
    ebase = wid * EC_B
    pltpu.sync_copy(src_h.at[pl.ds(ebase, EC_B)], sibuf)
    pltpu.sync_copy(dst_h.at[pl.ds(ebase, EC_B)], dibuf)
    for idxbuf, out_h in ((sibuf, Hs_h), (dibuf, Hd_h)):
        g1 = pltpu.async_copy(H_h.at[idxbuf.at[pl.ds(0, EH)]], hbuf, sem)
        g2 = pltpu.async_copy(H_h.at[idxbuf.at[pl.ds(EH, EH)]], hbuf2, sem2)
        g1.wait()
        pltpu.sync_copy(hbuf, out_h.at[pl.ds(ebase, EH)])
        g2.wait()
        pltpu.sync_copy(hbuf2, out_h.at[pl.ds(ebase + EH, EH)])


def _sc_scatter_gather(row, col, src, dst, H):
    return pl.kernel(
        _sc_scatter_gather_body,
        out_type=(
            jax.ShapeDtypeStruct((NPOS,), i32),
            jax.ShapeDtypeStruct((E, D), f32),
            jax.ShapeDtypeStruct((E, D), f32),
        ),
        mesh=_MESH,
        name="sc1_scatter_gather",
        scratch_types=[
            pltpu.VMEM((CH_B,), i32),
            pltpu.VMEM((CH_B,), i32),
            pltpu.VMEM((CH_B,), i32),
            pltpu.VMEM((CH_B,), i32),
            pltpu.VMEM((EC_B,), i32),
            pltpu.VMEM((EC_B,), i32),
            pltpu.VMEM((EC_B // 2, D), f32),
            pltpu.VMEM((EC_B // 2, D), f32),
            pltpu.SemaphoreType.DMA,
            pltpu.SemaphoreType.DMA,
        ],
    )(row, col, src, dst, H)


CH_D = NNZ // NS
ZB = 2048
NPS = N // NS


def _sc_dedup_body(row_h, col_h, v_h, e_h, T_h,
                   u_h, s_h,
                   rbuf, cbuf, pbuf, wbuf, vbuf, ubuf, zbuf, acc, sacc, sem):
    c = lax.axis_index("c")
    sid = lax.axis_index("s")
    base = sid * CH_D
    _zero_vmem(zbuf, ZB)

    @pl.when(c == 0)
    def _():
        for j in range(CH_D // ZB):
            pltpu.sync_copy(zbuf, acc.at[pl.ds(base + j * ZB, ZB)])
        pltpu.sync_copy(row_h.at[pl.ds(base, CH_D)], rbuf)
        pltpu.sync_copy(col_h.at[pl.ds(base, CH_D)], cbuf)

        def body(i, _):
            pbuf[pl.ds(i * L, L)] = rbuf[pl.ds(i * L, L)] * N + cbuf[pl.ds(i * L, L)]
            return 0

        lax.fori_loop(0, CH_D // L, body, 0)
        pltpu.async_copy(T_h.at[pbuf], wbuf, sem).wait()
        pltpu.sync_copy(v_h.at[pl.ds(base, CH_D)], vbuf)

    @pl.when(c == 1)
    def _():
        pltpu.sync_copy(zbuf.at[pl.ds(0, NPS)], sacc.at[pl.ds(sid * NPS, NPS)])
        pltpu.sync_copy(e_h.at[pl.ds(base, CH_D)], vbuf)
        pltpu.sync_copy(col_h.at[pl.ds(base, CH_D)], cbuf)

    plsc.subcore_barrier()

    @pl.when(c == 0)
    def _():
        pltpu.sync_copy(vbuf, acc.at[wbuf], add=True)

    @pl.when(c == 1)
    def _():
        pltpu.sync_copy(vbuf, sacc.at[cbuf], add=True)

    plsc.subcore_barrier()

    @pl.when(c == 0)
    def _():
        pltpu.async_copy(acc.at[wbuf], ubuf, sem).wait()
        pltpu.sync_copy(ubuf, u_h.at[pl.ds(base, CH_D)])

    @pl.when(c == 1)
    def _():
        pltpu.sync_copy(sacc.at[pl.ds(sid * NPS, NPS)], s_h.at[pl.ds(sid * NPS, NPS)])


def _sc_dedup(row, col, v, e_flat, T):
    return pl.kernel(
        _sc_dedup_body,
        out_type=(
            jax.ShapeDtypeStruct((NNZ,), f32),
            jax.ShapeDtypeStruct((N,), f32),
        ),
        mesh=_MESH,
        name="sc2_dedup",
        scratch_types=[
            pltpu.VMEM((CH_D,), i32),
            pltpu.VMEM((CH_D,), i32),
            pltpu.VMEM((CH_D,), i32),
            pltpu.VMEM((CH_D,), i32),
            pltpu.VMEM((CH_D,), f32),
            pltpu.VMEM((CH_D,), f32),
            pltpu.VMEM((ZB,), f32),
            pltpu.VMEM_SHARED((NNZ,), f32),
            pltpu.VMEM_SHARED((N,), f32),
            pltpu.SemaphoreType.DMA,
        ],
    )(row, col, v, e_flat, T)


EPT = E // NS
NG = NPS // L


def _sc_final_body(row_h, col_h, v_h, u_h, s2_h, src_h, hc_h, a0_h,
                   pw1_h, pb1_h, pw2_h, pb2_h,
                   out_h,
                   rbuf, cbuf, vbuf, ubuf, dbuf, s2gbuf, zerobuf,
                   tbuf, znbuf, w1buf, b1buf, w2buf, b2buf, a0buf,
                   sbuf, hcbuf, obuf, zgbuf, s2sp, tacc, zsp, sem):
    c = lax.axis_index("c")
    sid = lax.axis_index("s")
    base = sid * CH_D
    _zero_vmem(zerobuf, ZB)

    @pl.when(c == 0)
    def _():
        pltpu.sync_copy(zerobuf.at[pl.ds(0, NPS)], tacc.at[pl.ds(sid * NPS, NPS)])
        pltpu.sync_copy(s2_h.at[pl.ds(sid * NPS, NPS)], s2sp.at[pl.ds(sid * NPS, NPS)])
        pltpu.sync_copy(row_h.at[pl.ds(base, CH_D)], rbuf)
        pltpu.sync_copy(col_h.at[pl.ds(base, CH_D)], cbuf)
        pltpu.sync_copy(v_h.at[pl.ds(base, CH_D)], vbuf)
        pltpu.sync_copy(u_h.at[pl.ds(base, CH_D)], ubuf)

    plsc.subcore_barrier()

    @pl.when(c == 0)
    def _():
        pltpu.async_copy(s2sp.at[cbuf], s2gbuf, sem).wait()

        def body(i, _):
            sl = pl.ds(i * L, L)
            dbuf[sl] = vbuf[sl] * ubuf[sl] * s2gbuf[sl]
            return 0

        lax.fori_loop(0, CH_D // L, body, 0)
        pltpu.sync_copy(dbuf, tacc.at[rbuf], add=True)

    plsc.subcore_barrier()

    @pl.when(c == 0)
    def _():
        pltpu.sync_copy(tacc.at[pl.ds(sid * NPS, NPS)], tbuf)
        pltpu.sync_copy(pw1_h, w1buf)
        pltpu.sync_copy(pb1_h, b1buf)
        pltpu.sync_copy(pw2_h, w2buf)
        pltpu.sync_copy(pb2_h, b2buf)
        tv = [tbuf[pl.ds(g * L, L)] for g in range(NG)]
        accs = [jnp.zeros((L,), f32) for _ in range(NG)]

        def hbody(h, accs):
            w1h = w1buf[pl.ds(h, L)][0]
            b1h = b1buf[pl.ds(h, L)][0]
            w2h = w2buf[pl.ds(h, L)][0]
            return tuple(
                a + jnp.maximum(t * w1h + b1h, 0.0) * w2h
                for a, t in zip(accs, tv)
            )

        accs = lax.fori_loop(0, D, hbody, tuple(accs))
        b2v = b2buf[...]
        for g in range(NG):
            z = 1.0 / (1.0 + jnp.exp(-(accs[g] + b2v)))
            znbuf[pl.ds(g * L, L)] = z
        pltpu.sync_copy(znbuf, zsp.at[pl.ds(sid * NPS, NPS)])

    plsc.subcore_barrier()

    @pl.when(c == 0)
    def _():
        ebase = sid * EPT
        pltpu.sync_copy(src_h.at[pl.ds(ebase, EPT)], sbuf)
        pltpu.sync_copy(hc_h.at[pl.ds(ebase, EPT)], hcbuf)
        pltpu.sync_copy(a0_h, a0buf)
        pltpu.async_copy(zsp.at[sbuf], zgbuf, sem).wait()
        a0v = a0buf[...]

        def ebody(i, _):
            sl = pl.ds(i * L, L)
            obuf[sl] = a0v * zgbuf[sl] + hcbuf[sl]
            return 0

        lax.fori_loop(0, EPT // L, ebody, 0)
        pltpu.sync_copy(obuf, out_h.at[pl.ds(ebase, EPT)])


def _sc_final(row, col, v, u, s2, src, hc, a0vec, pw1, pb1, pw2, pb2):
    return pl.kernel(
        _sc_final_body,
        out_type=jax.ShapeDtypeStruct((E,), f32),
        mesh=_MESH,
        name="sc3_final",
        scratch_types=[
            pltpu.VMEM((CH_D,), i32),
            pltpu.VMEM((CH_D,), i32),
            pltpu.VMEM((CH_D,), f32),
            pltpu.VMEM((CH_D,), f32),
            pltpu.VMEM((CH_D,), f32),
            pltpu.VMEM((CH_D,), f32),
            pltpu.VMEM((ZB,), f32),
            pltpu.VMEM((NPS,), f32),
            pltpu.VMEM((NPS,), f32),
            pltpu.VMEM((D + L,), f32),
            pltpu.VMEM((D + L,), f32),
            pltpu.VMEM((D + L,), f32),
            pltpu.VMEM((L,), f32),
            pltpu.VMEM((L,), f32),
            pltpu.VMEM((EPT,), i32),
            pltpu.VMEM((EPT,), f32),
            pltpu.VMEM((EPT,), f32),
            pltpu.VMEM((EPT,), f32),
            pltpu.VMEM_SHARED((N,), f32),
            pltpu.VMEM_SHARED((N,), f32),
            pltpu.VMEM_SHARED((N,), f32),
            pltpu.SemaphoreType.DMA,
        ],
    )(row, col, v, u, s2, src, hc, a0vec, pw1, pb1, pw2, pb2)


def kernel(edges, H, overlap_row, overlap_col, overlap_values,
           f_edge_w1, f_edge_b1, f_edge_w2, f_edge_b2,
           f_node_w1, f_node_b1, f_node_w2, f_node_b2,
           psi3_w1, psi3_b1, psi3_w2, psi3_b2, alpha):
    src = edges[:, 0].astype(i32)
    dst = edges[:, 1].astype(i32)
    row = overlap_row.astype(i32)
    col = overlap_col.astype(i32)
    v = overlap_values.astype(f32)

    e = _edge_mlp(v[:, None], f_edge_w1, f_edge_b1[None, :], f_edge_w2,
                  f_edge_b2[None, :])[:, 0]

    T, hs, hd = _sc_scatter_gather(row, col, src, dst, H)

    u, s = _sc_dedup(row, col, v, e, T)

    alpha_pad = jnp.zeros((1, D), f32).at[0, :2].set(alpha.astype(f32))
    hc2, a0vec2 = _edge_dot(hs, hd, alpha_pad)
    hc = hc2[:, 0]
    a0vec = a0vec2[0]

    s2 = _node_mlp_sq(s[:, None], f_node_w1, f_node_b1[None, :], f_node_w2,
                      f_node_b2[None, :])[:, 0]

    pad = lambda x: jnp.pad(x.reshape(D), (0, L))
    out = _sc_final(row, col, v, u, s2, src, hc, a0vec,
                    pad(psi3_w1), pad(psi3_b1), pad(psi3_w2),
                    jnp.broadcast_to(psi3_b2, (L,)))

    return out[:, None]
